# baseline (speedup 1.0000x reference)
import functools
import jax, jax.numpy as jnp
from jax import lax
from jax.experimental import pallas as pl
from jax.experimental.pallas import tpu as pltpu

R = 128
N = 32768
BN = 16384
SL = 8
NCH = R // SL


def _tc_argmax_body(x_ref, o_ref):
    vmax = x_ref[0:SL, :]
    vchunk = jnp.zeros((SL, BN), jnp.int32)
    for c in range(1, NCH):
        v = x_ref[SL * c:SL * (c + 1), :]
        p = v > vmax
        vmax = jnp.where(p, v, vmax)
        vchunk = jnp.where(p, jnp.int32(c), vchunk)
    m = jnp.max(vmax, axis=0)
    srow = lax.broadcasted_iota(jnp.int32, (SL, BN), 0)
    cand = jnp.where(vmax == m[None, :],
                     (vchunk << 3) | srow,
                     jnp.int32(R))
    o_ref[...] = jnp.min(cand, axis=0)


def kernel(x):
    out = pl.pallas_call(
        _tc_argmax_body,
        out_shape=jax.ShapeDtypeStruct((N,), jnp.int32),
        grid=(N // BN,),
        in_specs=[pl.BlockSpec((R, BN), lambda i: (0, i))],
        out_specs=pl.BlockSpec((BN,), lambda i: (i,)),
    )(x)
    return out.astype(jnp.int64)
